# Initial kernel scaffold; baseline (speedup 1.0000x reference)
import jax
import jax.numpy as jnp
from jax.experimental import pallas as pl


def kernel(x, edge_index, W_self0, W_neigh0, b0, W_self1, W_neigh1, b1, W_self2, W_neigh2, b2):
    raise NotImplementedError("write your pallas kernel here")


# speedup vs baseline: 4.9103x; 4.9103x over previous
import functools

import jax
import jax.numpy as jnp
from jax import lax
from jax.experimental import pallas as pl
from jax.experimental.pallas import tpu as pltpu
from jax.experimental.pallas import tpu_sc as plsc

N = 10000
E = 160000
D = 256
H = 256
C = 64

NSUB = 16
K = 80
CHUNKS = E // (K * NSUB)
GC = 25
NG = CHUNKS // GC
RPT = 624
RPT_LAST = N - 15 * RPT

BR = 400
GRID = N // BR


def _mm_first_body(x_ref, ws_ref, wn_ref, b_ref, s_ref, pa_ref, pb_ref):
    h = x_ref[...]
    s_ref[...] = jnp.dot(h, ws_ref[...], preferred_element_type=jnp.float32) + b_ref[...]
    p = jnp.dot(h, wn_ref[...], preferred_element_type=jnp.float32)
    f2 = p.shape[1] // 2
    pa_ref[...] = p[:, :f2]
    pb_ref[...] = p[:, f2:]


def _mm_mid_body(s_prev_ref, agga_ref, aggb_ref, dega_ref, degb_ref,
                 ws_ref, wn_ref, b_ref, s_ref, pa_ref, pb_ref):
    inv = 1.0 / jnp.maximum(dega_ref[...][:, :1] + degb_ref[...][:, :1], 1.0)
    h = s_prev_ref[...] + jnp.concatenate(
        [agga_ref[...] * inv, aggb_ref[...] * inv], axis=1)
    h = jnp.maximum(h, 0.0)
    s_ref[...] = jnp.dot(h, ws_ref[...], preferred_element_type=jnp.float32) + b_ref[...]
    p = jnp.dot(h, wn_ref[...], preferred_element_type=jnp.float32)
    f2 = p.shape[1] // 2
    pa_ref[...] = p[:, :f2]
    pb_ref[...] = p[:, f2:]


def _mm_last_body(s_prev_ref, agga_ref, aggb_ref, dega_ref, degb_ref,
                  ws_ref, wn_ref, b_ref, s_ref, p_ref):
    inv = 1.0 / jnp.maximum(dega_ref[...][:, :1] + degb_ref[...][:, :1], 1.0)
    h = s_prev_ref[...] + jnp.concatenate(
        [agga_ref[...] * inv, aggb_ref[...] * inv], axis=1)
    h = jnp.maximum(h, 0.0)
    s_ref[...] = jnp.dot(h, ws_ref[...], preferred_element_type=jnp.float32) + b_ref[...]
    p_ref[...] = jnp.dot(h, wn_ref[...], preferred_element_type=jnp.float32)


def _final_body(s_ref, agga_ref, aggb_ref, dega_ref, degb_ref, o_ref):
    inv = 1.0 / jnp.maximum(dega_ref[...][:, :1] + degb_ref[...][:, :1], 1.0)
    o_ref[...] = s_ref[...] + (agga_ref[...][:, :C] + aggb_ref[...][:, :C]) * inv


def _row_spec(w):
    return pl.BlockSpec((BR, w), lambda i: (i, 0))


def _full_spec(r, c):
    return pl.BlockSpec((r, c), lambda i: (0, 0))


def _mm_first(x, ws, wn, b, fout):
    return pl.pallas_call(
        _mm_first_body,
        grid=(GRID,),
        in_specs=[_row_spec(D), _full_spec(D, fout), _full_spec(D, fout),
                  _full_spec(1, fout)],
        out_specs=[_row_spec(fout), _row_spec(fout // 2), _row_spec(fout // 2)],
        out_shape=[jax.ShapeDtypeStruct((N, fout), jnp.float32),
                   jax.ShapeDtypeStruct((N, fout // 2), jnp.float32),
                   jax.ShapeDtypeStruct((N, fout // 2), jnp.float32)],
    )(x, ws, wn, b.reshape(1, fout))


def _mm_mid(s_prev, agga, aggb, dega, degb, ws, wn, b, fin, fout):
    return pl.pallas_call(
        _mm_mid_body,
        grid=(GRID,),
        in_specs=[_row_spec(fin), _row_spec(fin // 2), _row_spec(fin // 2),
                  _row_spec(128), _row_spec(128),
                  _full_spec(fin, fout), _full_spec(fin, fout),
                  _full_spec(1, fout)],
        out_specs=[_row_spec(fout), _row_spec(fout // 2), _row_spec(fout // 2)],
        out_shape=[jax.ShapeDtypeStruct((N, fout), jnp.float32),
                   jax.ShapeDtypeStruct((N, fout // 2), jnp.float32),
                   jax.ShapeDtypeStruct((N, fout // 2), jnp.float32)],
    )(s_prev, agga, aggb, dega, degb, ws, wn, b.reshape(1, fout))


def _mm_last(s_prev, agga, aggb, dega, degb, ws, wn_pad, b, fin, fout):
    return pl.pallas_call(
        _mm_last_body,
        grid=(GRID,),
        in_specs=[_row_spec(fin), _row_spec(fin // 2), _row_spec(fin // 2),
                  _row_spec(128), _row_spec(128),
                  _full_spec(fin, fout), _full_spec(fin, 128),
                  _full_spec(1, fout)],
        out_specs=[_row_spec(fout), _row_spec(128)],
        out_shape=[jax.ShapeDtypeStruct((N, fout), jnp.float32),
                   jax.ShapeDtypeStruct((N, 128), jnp.float32)],
    )(s_prev, agga, aggb, dega, degb, ws, wn_pad, b.reshape(1, fout))


def _final(s, agga, aggb, dega, degb, fout):
    return pl.pallas_call(
        _final_body,
        grid=(GRID,),
        in_specs=[_row_spec(fout), _row_spec(128), _row_spec(128),
                  _row_spec(128), _row_spec(128)],
        out_specs=_row_spec(fout),
        out_shape=jax.ShapeDtypeStruct((N, fout), jnp.float32),
    )(s, agga, aggb, dega, degb)


def _fill_zero(buf, nrows, ncols):
    @pl.loop(0, nrows)
    def _(r):
        for jj in range(ncols // 16):
            buf[r, pl.ds(jj * 16, 16)] = jnp.zeros((16,), jnp.float32)


def _pieces(n, cap):
    off = 0
    out = []
    while off < n:
        m = min(cap, n - off)
        out.append((off, m))
        off += m
    return out


def _each_rows(s, fn):
    @pl.when(s < 15)
    def _():
        fn(pl.multiple_of(s * RPT, 8), RPT)

    @pl.when(s == 15)
    def _():
        fn(15 * RPT, RPT_LAST)


def _zero_spmem(s, sp, stage, cap):
    def z(base, n):
        for off, m in _pieces(n, cap):
            pltpu.sync_copy(stage.at[pl.ds(0, m)], sp.at[pl.ds(base + off, m)])
    _each_rows(s, z)


def _spmem_to_hbm(s, sp, hbm, stage, cap):
    def w(base, n):
        for off, m in _pieces(n, cap):
            pltpu.sync_copy(sp.at[pl.ds(base + off, m)], stage.at[pl.ds(0, m)])
            pltpu.sync_copy(stage.at[pl.ds(0, m)], hbm.at[pl.ds(base + off, m)])
    _each_rows(s, w)


@functools.lru_cache(maxsize=None)
def _make_segsum(fh):
    mesh = plsc.VectorSubcoreMesh(core_axis_name="c", subcore_axis_name="s")
    out_type = [jax.ShapeDtypeStruct((N, fh), jnp.float32),
                jax.ShapeDtypeStruct((N, fh), jnp.float32)]
    scratch = [
        pltpu.VMEM((GC, K), jnp.int32),
        pltpu.VMEM((GC, K), jnp.int32),
        pltpu.VMEM((K, fh), jnp.float32),
        pltpu.VMEM_SHARED((N, fh), jnp.float32),
        pltpu.SemaphoreType.DMA,
    ]

    def body(pa_hbm, pb_hbm, src_hbm, dst_hbm,
             agga_hbm, aggb_hbm, src_v, dst_v, rows_v, acc, sem):
        c = lax.axis_index("c")
        s = lax.axis_index("s")

        _fill_zero(rows_v, K, fh)
        _zero_spmem(s, acc, rows_v, K)
        plsc.subcore_barrier()

        @pl.loop(0, NG)
        def _(g):
            pltpu.sync_copy(src_hbm.at[s, g], src_v)
            pltpu.sync_copy(dst_hbm.at[s, g], dst_v)

            def inner(p_hbm):
                @pl.loop(0, GC)
                def _(j):
                    pltpu.async_copy(p_hbm.at[src_v.at[j]], rows_v, sem).wait()
                    pltpu.sync_copy(rows_v, acc.at[dst_v.at[j]], add=True)

            @pl.when(c == 0)
            def _():
                inner(pa_hbm)

            @pl.when(c == 1)
            def _():
                inner(pb_hbm)

        plsc.subcore_barrier()

        @pl.when(c == 0)
        def _():
            _spmem_to_hbm(s, acc, agga_hbm, rows_v, K)

        @pl.when(c == 1)
        def _():
            _spmem_to_hbm(s, acc, aggb_hbm, rows_v, K)

    return pl.kernel(body, out_type=out_type, mesh=mesh, scratch_types=scratch)


@functools.lru_cache(maxsize=None)
def _make_deg():
    mesh = plsc.VectorSubcoreMesh(core_axis_name="c", subcore_axis_name="s")
    out_type = [jax.ShapeDtypeStruct((N, 128), jnp.float32),
                jax.ShapeDtypeStruct((N, 128), jnp.float32)]
    scratch = [
        pltpu.VMEM((GC, K2), jnp.int32),
        pltpu.VMEM((K2, 128), jnp.float32),
        pltpu.VMEM_SHARED((N, 128), jnp.float32),
    ]

    def body(dst_hbm, dega_hbm, degb_hbm, dst_v, ones_v, dacc):
        c = lax.axis_index("c")
        s = lax.axis_index("s")

        _fill_zero(ones_v, K2, 128)
        _zero_spmem(s, dacc, ones_v, K2)

        @pl.loop(0, K2)
        def _(r):
            for jj in range(8):
                ones_v[r, pl.ds(jj * 16, 16)] = jnp.ones((16,), jnp.float32)

        plsc.subcore_barrier()

        @pl.loop(0, NG)
        def _(g):
            pltpu.sync_copy(dst_hbm.at[c, s, g], dst_v)

            @pl.loop(0, GC)
            def _(j):
                pltpu.sync_copy(ones_v, dacc.at[dst_v.at[j]], add=True)

        plsc.subcore_barrier()

        @pl.when(c == 0)
        def _():
            _spmem_to_hbm(s, dacc, dega_hbm, ones_v, K2)

        @pl.when(c == 1)
        def _():
            _spmem_to_hbm(s, dacc, degb_hbm, ones_v, K2)

    return pl.kernel(body, out_type=out_type, mesh=mesh, scratch_types=scratch)


K2 = 40


@functools.lru_cache(maxsize=None)
def _make_segsum_split():
    mesh = plsc.VectorSubcoreMesh(core_axis_name="c", subcore_axis_name="s")
    out_type = [jax.ShapeDtypeStruct((N, 128), jnp.float32),
                jax.ShapeDtypeStruct((N, 128), jnp.float32)]
    scratch = [
        pltpu.VMEM((GC, K2), jnp.int32),
        pltpu.VMEM((GC, K2), jnp.int32),
        pltpu.VMEM((K2, 128), jnp.float32),
        pltpu.VMEM_SHARED((N, 128), jnp.float32),
        pltpu.SemaphoreType.DMA,
    ]

    def body(p_hbm, src_hbm, dst_hbm,
             agga_hbm, aggb_hbm, src_v, dst_v, rows_v, acc, sem):
        c = lax.axis_index("c")
        s = lax.axis_index("s")

        _fill_zero(rows_v, K2, 128)
        _zero_spmem(s, acc, rows_v, K2)
        plsc.subcore_barrier()

        @pl.loop(0, NG)
        def _(g):
            pltpu.sync_copy(src_hbm.at[c, s, g], src_v)
            pltpu.sync_copy(dst_hbm.at[c, s, g], dst_v)

            @pl.loop(0, GC)
            def _(j):
                pltpu.async_copy(p_hbm.at[src_v.at[j]], rows_v, sem).wait()
                pltpu.sync_copy(rows_v, acc.at[dst_v.at[j]], add=True)

        plsc.subcore_barrier()

        @pl.when(c == 0)
        def _():
            _spmem_to_hbm(s, acc, agga_hbm, rows_v, K2)

        @pl.when(c == 1)
        def _():
            _spmem_to_hbm(s, acc, aggb_hbm, rows_v, K2)

    return pl.kernel(body, out_type=out_type, mesh=mesh, scratch_types=scratch)


def kernel(x, edge_index, W_self0, W_neigh0, b0, W_self1, W_neigh1, b1,
           W_self2, W_neigh2, b2):
    src = edge_index[0]
    dst = edge_index[1]
    src2 = src.reshape(NSUB, NG, GC, K)
    dst2 = dst.reshape(NSUB, NG, GC, K)
    src5 = src.reshape(2, NSUB, NG, GC, K2)
    dst5 = dst.reshape(2, NSUB, NG, GC, K2)
    wn2_pad = jnp.pad(W_neigh2, ((0, 0), (0, 128 - C)))

    dega, degb = _make_deg()(dst5)
    s0, p0a, p0b = _mm_first(x, W_self0, W_neigh0, b0, H)
    agg0a, agg0b = _make_segsum(H // 2)(p0a, p0b, src2, dst2)
    s1, p1a, p1b = _mm_mid(s0, agg0a, agg0b, dega, degb,
                           W_self1, W_neigh1, b1, H, H)
    agg1a, agg1b = _make_segsum(H // 2)(p1a, p1b, src2, dst2)
    s2, p2 = _mm_last(s1, agg1a, agg1b, dega, degb, W_self2, wn2_pad, b2, H, C)
    agg2a, agg2b = _make_segsum_split()(p2, src5, dst5)
    return _final(s2, agg2a, agg2b, dega, degb, C)


# speedup vs baseline: 6.7635x; 1.3774x over previous
import functools

import jax
import jax.numpy as jnp
from jax import lax
from jax.experimental import pallas as pl
from jax.experimental.pallas import tpu as pltpu
from jax.experimental.pallas import tpu_sc as plsc

N = 10000
E = 160000
D = 256
H = 256
C = 64

NSUB = 16
K = 100
CHUNKS = E // (K * NSUB)
GC = 20
NG = CHUNKS // GC
RPT = 624
RPT_LAST = N - 15 * RPT

BR = 400
GRID = N // BR


def _mm_first_body(x_ref, ws_ref, wn_ref, b_ref, s_ref, pa_ref, pb_ref):
    h = x_ref[...]
    s_ref[...] = jnp.dot(h, ws_ref[...], preferred_element_type=jnp.float32) + b_ref[...]
    p = jnp.dot(h, wn_ref[...], preferred_element_type=jnp.float32)
    f2 = p.shape[1] // 2
    pa_ref[...] = p[:, :f2]
    pb_ref[...] = p[:, f2:]


def _mm_mid_body(s_prev_ref, agga_ref, aggb_ref, dega_ref, degb_ref,
                 ws_ref, wn_ref, b_ref, s_ref, pa_ref, pb_ref):
    inv = 1.0 / jnp.maximum(dega_ref[...][:, :1] + degb_ref[...][:, :1], 1.0)
    h = s_prev_ref[...] + jnp.concatenate(
        [agga_ref[...] * inv, aggb_ref[...] * inv], axis=1)
    h = jnp.maximum(h, 0.0)
    s_ref[...] = jnp.dot(h, ws_ref[...], preferred_element_type=jnp.float32) + b_ref[...]
    p = jnp.dot(h, wn_ref[...], preferred_element_type=jnp.float32)
    f2 = p.shape[1] // 2
    pa_ref[...] = p[:, :f2]
    pb_ref[...] = p[:, f2:]


def _mm_last_body(s_prev_ref, agga_ref, aggb_ref, dega_ref, degb_ref,
                  ws_ref, wn_ref, b_ref, s_ref, p_ref):
    inv = 1.0 / jnp.maximum(dega_ref[...][:, :1] + degb_ref[...][:, :1], 1.0)
    h = s_prev_ref[...] + jnp.concatenate(
        [agga_ref[...] * inv, aggb_ref[...] * inv], axis=1)
    h = jnp.maximum(h, 0.0)
    s_ref[...] = jnp.dot(h, ws_ref[...], preferred_element_type=jnp.float32) + b_ref[...]
    p_ref[...] = jnp.dot(h, wn_ref[...], preferred_element_type=jnp.float32)


def _final_body(s_ref, agga_ref, aggb_ref, dega_ref, degb_ref, o_ref):
    inv = 1.0 / jnp.maximum(dega_ref[...][:, :1] + degb_ref[...][:, :1], 1.0)
    o_ref[...] = s_ref[...] + (agga_ref[...][:, :C] + aggb_ref[...][:, :C]) * inv


def _row_spec(w):
    return pl.BlockSpec((BR, w), lambda i: (i, 0))


def _full_spec(r, c):
    return pl.BlockSpec((r, c), lambda i: (0, 0))


def _mm_first(x, ws, wn, b, fout):
    return pl.pallas_call(
        _mm_first_body,
        grid=(GRID,),
        in_specs=[_row_spec(D), _full_spec(D, fout), _full_spec(D, fout),
                  _full_spec(1, fout)],
        out_specs=[_row_spec(fout), _row_spec(fout // 2), _row_spec(fout // 2)],
        out_shape=[jax.ShapeDtypeStruct((N, fout), jnp.float32),
                   jax.ShapeDtypeStruct((N, fout // 2), jnp.float32),
                   jax.ShapeDtypeStruct((N, fout // 2), jnp.float32)],
    )(x, ws, wn, b.reshape(1, fout))


def _mm_mid(s_prev, agga, aggb, dega, degb, ws, wn, b, fin, fout):
    return pl.pallas_call(
        _mm_mid_body,
        grid=(GRID,),
        in_specs=[_row_spec(fin), _row_spec(fin // 2), _row_spec(fin // 2),
                  _row_spec(128), _row_spec(128),
                  _full_spec(fin, fout), _full_spec(fin, fout),
                  _full_spec(1, fout)],
        out_specs=[_row_spec(fout), _row_spec(fout // 2), _row_spec(fout // 2)],
        out_shape=[jax.ShapeDtypeStruct((N, fout), jnp.float32),
                   jax.ShapeDtypeStruct((N, fout // 2), jnp.float32),
                   jax.ShapeDtypeStruct((N, fout // 2), jnp.float32)],
    )(s_prev, agga, aggb, dega, degb, ws, wn, b.reshape(1, fout))


def _mm_last(s_prev, agga, aggb, dega, degb, ws, wn_pad, b, fin, fout):
    return pl.pallas_call(
        _mm_last_body,
        grid=(GRID,),
        in_specs=[_row_spec(fin), _row_spec(fin // 2), _row_spec(fin // 2),
                  _row_spec(128), _row_spec(128),
                  _full_spec(fin, fout), _full_spec(fin, 128),
                  _full_spec(1, fout)],
        out_specs=[_row_spec(fout), _row_spec(128)],
        out_shape=[jax.ShapeDtypeStruct((N, fout), jnp.float32),
                   jax.ShapeDtypeStruct((N, 128), jnp.float32)],
    )(s_prev, agga, aggb, dega, degb, ws, wn_pad, b.reshape(1, fout))


def _final(s, agga, aggb, dega, degb, fout):
    return pl.pallas_call(
        _final_body,
        grid=(GRID,),
        in_specs=[_row_spec(fout), _row_spec(128), _row_spec(128),
                  _row_spec(128), _row_spec(128)],
        out_specs=_row_spec(fout),
        out_shape=jax.ShapeDtypeStruct((N, fout), jnp.float32),
    )(s, agga, aggb, dega, degb)


def _fill_zero(buf, nrows, ncols):
    @pl.loop(0, nrows)
    def _(r):
        for jj in range(ncols // 16):
            buf[r, pl.ds(jj * 16, 16)] = jnp.zeros((16,), jnp.float32)


def _pieces(n, cap):
    off = 0
    out = []
    while off < n:
        m = min(cap, n - off)
        out.append((off, m))
        off += m
    return out


def _each_rows(s, fn):
    @pl.when(s < 15)
    def _():
        fn(pl.multiple_of(s * RPT, 8), RPT)

    @pl.when(s == 15)
    def _():
        fn(15 * RPT, RPT_LAST)


def _zero_spmem(s, sp, stage, cap):
    def z(base, n):
        for off, m in _pieces(n, cap):
            pltpu.sync_copy(stage.at[pl.ds(0, m)], sp.at[pl.ds(base + off, m)])
    _each_rows(s, z)


def _spmem_to_hbm(s, sp, hbm, stage, cap):
    def w(base, n):
        for off, m in _pieces(n, cap):
            pltpu.sync_copy(sp.at[pl.ds(base + off, m)], stage.at[pl.ds(0, m)])
            pltpu.sync_copy(stage.at[pl.ds(0, m)], hbm.at[pl.ds(base + off, m)])
    _each_rows(s, w)


@functools.lru_cache(maxsize=None)
def _make_segsum(fh):
    mesh = plsc.VectorSubcoreMesh(core_axis_name="c", subcore_axis_name="s")
    out_type = [jax.ShapeDtypeStruct((N, fh), jnp.float32),
                jax.ShapeDtypeStruct((N, fh), jnp.float32)]
    scratch = [
        pltpu.VMEM((GC, K), jnp.int32),
        pltpu.VMEM((GC, K), jnp.int32),
        pltpu.VMEM((K, fh), jnp.float32),
        pltpu.VMEM((K, fh), jnp.float32),
        pltpu.VMEM_SHARED((N, fh), jnp.float32),
        pltpu.SemaphoreType.DMA,
        pltpu.SemaphoreType.DMA,
    ]

    def body(pa_hbm, pb_hbm, src_hbm, dst_hbm, dummy_hbm,
             agga_hbm, aggb_hbm, src_v, dst_v, r0, r1, acc, sg0, sg1):
        c = lax.axis_index("c")
        s = lax.axis_index("s")

        _fill_zero(r0, K, fh)
        _zero_spmem(s, acc, r0, 80)
        plsc.subcore_barrier()

        @pl.loop(0, NG)
        def _(g):
            pltpu.sync_copy(src_hbm.at[s, g], src_v)
            pltpu.sync_copy(dst_hbm.at[s, g], dst_v)

            def inner(p_hbm):
                def gwait(buf, sem):
                    pltpu.make_async_copy(dummy_hbm, buf, sem).wait()

                pltpu.async_copy(p_hbm.at[src_v.at[0]], r0, sg0)

                @pl.loop(0, GC, step=2)
                def _(j):
                    gwait(r0, sg0)
                    pltpu.async_copy(p_hbm.at[src_v.at[j + 1]], r1, sg1)
                    pltpu.sync_copy(r0, acc.at[dst_v.at[j]], add=True)
                    gwait(r1, sg1)

                    @pl.when(j + 2 < GC)
                    def _():
                        pltpu.async_copy(p_hbm.at[src_v.at[j + 2]], r0, sg0)

                    pltpu.sync_copy(r1, acc.at[dst_v.at[j + 1]], add=True)

            @pl.when(c == 0)
            def _():
                inner(pa_hbm)

            @pl.when(c == 1)
            def _():
                inner(pb_hbm)

        plsc.subcore_barrier()

        @pl.when(c == 0)
        def _():
            _spmem_to_hbm(s, acc, agga_hbm, r0, 80)

        @pl.when(c == 1)
        def _():
            _spmem_to_hbm(s, acc, aggb_hbm, r0, 80)

    return pl.kernel(body, out_type=out_type, mesh=mesh, scratch_types=scratch)


@functools.lru_cache(maxsize=None)
def _make_deg():
    mesh = plsc.VectorSubcoreMesh(core_axis_name="c", subcore_axis_name="s")
    out_type = [jax.ShapeDtypeStruct((N, 128), jnp.float32),
                jax.ShapeDtypeStruct((N, 128), jnp.float32)]
    scratch = [
        pltpu.VMEM((GC2, K2), jnp.int32),
        pltpu.VMEM((K2, 128), jnp.float32),
        pltpu.VMEM_SHARED((N, 128), jnp.float32),
        pltpu.SemaphoreType.DMA,
    ]

    def body(dst_hbm, dummy_hbm, dega_hbm, degb_hbm, dst_v, ones_v, dacc, ss):
        c = lax.axis_index("c")
        s = lax.axis_index("s")

        _fill_zero(ones_v, K2, 128)
        _zero_spmem(s, dacc, ones_v, 80)

        @pl.loop(0, K2)
        def _(r):
            for jj in range(8):
                ones_v[r, pl.ds(jj * 16, 16)] = jnp.ones((16,), jnp.float32)

        plsc.subcore_barrier()

        @pl.loop(0, NG2)
        def _(g):
            pltpu.sync_copy(dst_hbm.at[c, s, g], dst_v)

            @pl.loop(0, GC2)
            def _(j):
                pltpu.async_copy(ones_v, dacc.at[dst_v.at[j]], ss, add=True)

            @pl.loop(0, GC2)
            def _(j):
                pltpu.make_async_copy(dummy_hbm, ones_v, ss).wait()

        plsc.subcore_barrier()

        @pl.when(c == 0)
        def _():
            _spmem_to_hbm(s, dacc, dega_hbm, ones_v, 80)

        @pl.when(c == 1)
        def _():
            _spmem_to_hbm(s, dacc, degb_hbm, ones_v, 80)

    return pl.kernel(body, out_type=out_type, mesh=mesh, scratch_types=scratch)


K2 = 100
GC2 = 10
NG2 = E // (2 * NSUB * K2 * GC2)


@functools.lru_cache(maxsize=None)
def _make_segsum_split():
    mesh = plsc.VectorSubcoreMesh(core_axis_name="c", subcore_axis_name="s")
    out_type = [jax.ShapeDtypeStruct((N, 128), jnp.float32),
                jax.ShapeDtypeStruct((N, 128), jnp.float32)]
    scratch = [
        pltpu.VMEM((GC2, K2), jnp.int32),
        pltpu.VMEM((GC2, K2), jnp.int32),
        pltpu.VMEM((K2, 128), jnp.float32),
        pltpu.VMEM((K2, 128), jnp.float32),
        pltpu.VMEM_SHARED((N, 128), jnp.float32),
        pltpu.SemaphoreType.DMA,
        pltpu.SemaphoreType.DMA,
    ]

    def body(p_hbm, src_hbm, dst_hbm, dummy_hbm,
             agga_hbm, aggb_hbm, src_v, dst_v, r0, r1, acc, sg0, sg1):
        c = lax.axis_index("c")
        s = lax.axis_index("s")

        _fill_zero(r0, K2, 128)
        _zero_spmem(s, acc, r0, 80)
        plsc.subcore_barrier()

        @pl.loop(0, NG2)
        def _(g):
            pltpu.sync_copy(src_hbm.at[c, s, g], src_v)
            pltpu.sync_copy(dst_hbm.at[c, s, g], dst_v)

            def gwait(buf, sem):
                pltpu.make_async_copy(dummy_hbm, buf, sem).wait()

            pltpu.async_copy(p_hbm.at[src_v.at[0]], r0, sg0)

            @pl.loop(0, GC2, step=2)
            def _(j):
                gwait(r0, sg0)
                pltpu.async_copy(p_hbm.at[src_v.at[j + 1]], r1, sg1)
                pltpu.sync_copy(r0, acc.at[dst_v.at[j]], add=True)
                gwait(r1, sg1)

                @pl.when(j + 2 < GC2)
                def _():
                    pltpu.async_copy(p_hbm.at[src_v.at[j + 2]], r0, sg0)

                pltpu.sync_copy(r1, acc.at[dst_v.at[j + 1]], add=True)

        plsc.subcore_barrier()

        @pl.when(c == 0)
        def _():
            _spmem_to_hbm(s, acc, agga_hbm, r0, 80)

        @pl.when(c == 1)
        def _():
            _spmem_to_hbm(s, acc, aggb_hbm, r0, 80)

    return pl.kernel(body, out_type=out_type, mesh=mesh, scratch_types=scratch)


def kernel(x, edge_index, W_self0, W_neigh0, b0, W_self1, W_neigh1, b1,
           W_self2, W_neigh2, b2):
    src = edge_index[0]
    dst = edge_index[1]
    src2 = src.reshape(NSUB, NG, GC, K)
    dst2 = dst.reshape(NSUB, NG, GC, K)
    src5 = src.reshape(2, NSUB, NG2, GC2, K2)
    dst5 = dst.reshape(2, NSUB, NG2, GC2, K2)
    wn2_pad = jnp.pad(W_neigh2, ((0, 0), (0, 128 - C)))

    dummy = jnp.zeros((K, 128), jnp.float32)
    dega, degb = _make_deg()(dst5, dummy)
    s0, p0a, p0b = _mm_first(x, W_self0, W_neigh0, b0, H)
    agg0a, agg0b = _make_segsum(H // 2)(p0a, p0b, src2, dst2, dummy)
    s1, p1a, p1b = _mm_mid(s0, agg0a, agg0b, dega, degb,
                           W_self1, W_neigh1, b1, H, H)
    agg1a, agg1b = _make_segsum(H // 2)(p1a, p1b, src2, dst2, dummy)
    s2, p2 = _mm_last(s1, agg1a, agg1b, dega, degb, W_self2, wn2_pad, b2, H, C)
    agg2a, agg2b = _make_segsum_split()(p2, src5, dst5, dummy)
    return _final(s2, agg2a, agg2b, dega, degb, C)


# speedup vs baseline: 7.8764x; 1.1646x over previous
import functools

import jax
import jax.numpy as jnp
from jax import lax
from jax.experimental import pallas as pl
from jax.experimental.pallas import tpu as pltpu
from jax.experimental.pallas import tpu_sc as plsc

N = 10000
E = 160000
D = 256
H = 256
C = 64

NSUB = 16
K = 125
CHUNKS = E // (K * NSUB)
GC = 20
NG = CHUNKS // GC
RPT = 624
RPT_LAST = N - 15 * RPT

BR = 2000
GRID = N // BR


def _mm_first_body(x_ref, ws_ref, wn_ref, b_ref, s_ref, pa_ref, pb_ref):
    h = x_ref[...]
    s_ref[...] = jnp.dot(h, ws_ref[...], preferred_element_type=jnp.float32) + b_ref[...]
    p = jnp.dot(h, wn_ref[...], preferred_element_type=jnp.float32)
    f2 = p.shape[1] // 2
    pa_ref[...] = p[:, :f2]
    pb_ref[...] = p[:, f2:]


def _mm_mid_body(s_prev_ref, agga_ref, aggb_ref, dega_ref, degb_ref,
                 ws_ref, wn_ref, b_ref, s_ref, pa_ref, pb_ref):
    inv = 1.0 / jnp.maximum(dega_ref[...][:, :1] + degb_ref[...][:, :1], 1.0)
    h = s_prev_ref[...] + jnp.concatenate(
        [agga_ref[...] * inv, aggb_ref[...] * inv], axis=1)
    h = jnp.maximum(h, 0.0)
    s_ref[...] = jnp.dot(h, ws_ref[...], preferred_element_type=jnp.float32) + b_ref[...]
    p = jnp.dot(h, wn_ref[...], preferred_element_type=jnp.float32)
    f2 = p.shape[1] // 2
    pa_ref[...] = p[:, :f2]
    pb_ref[...] = p[:, f2:]


def _mm_last_body(s_prev_ref, agga_ref, aggb_ref, dega_ref, degb_ref,
                  ws_ref, wn_ref, b_ref, s_ref, p_ref):
    inv = 1.0 / jnp.maximum(dega_ref[...][:, :1] + degb_ref[...][:, :1], 1.0)
    h = s_prev_ref[...] + jnp.concatenate(
        [agga_ref[...] * inv, aggb_ref[...] * inv], axis=1)
    h = jnp.maximum(h, 0.0)
    s_ref[...] = jnp.dot(h, ws_ref[...], preferred_element_type=jnp.float32) + b_ref[...]
    p_ref[...] = jnp.dot(h, wn_ref[...], preferred_element_type=jnp.float32)


def _final_body(s_ref, agga_ref, aggb_ref, dega_ref, degb_ref, o_ref):
    inv = 1.0 / jnp.maximum(dega_ref[...][:, :1] + degb_ref[...][:, :1], 1.0)
    o_ref[...] = s_ref[...] + (agga_ref[...][:, :C] + aggb_ref[...][:, :C]) * inv


def _row_spec(w):
    return pl.BlockSpec((BR, w), lambda i: (i, 0))


def _full_spec(r, c):
    return pl.BlockSpec((r, c), lambda i: (0, 0))


def _mm_first(x, ws, wn, b, fout):
    return pl.pallas_call(
        _mm_first_body,
        grid=(GRID,),
        in_specs=[_row_spec(D), _full_spec(D, fout), _full_spec(D, fout),
                  _full_spec(1, fout)],
        out_specs=[_row_spec(fout), _row_spec(fout // 2), _row_spec(fout // 2)],
        out_shape=[jax.ShapeDtypeStruct((N, fout), jnp.float32),
                   jax.ShapeDtypeStruct((N, fout // 2), jnp.float32),
                   jax.ShapeDtypeStruct((N, fout // 2), jnp.float32)],
    )(x, ws, wn, b.reshape(1, fout))


def _mm_mid(s_prev, agga, aggb, dega, degb, ws, wn, b, fin, fout):
    return pl.pallas_call(
        _mm_mid_body,
        grid=(GRID,),
        in_specs=[_row_spec(fin), _row_spec(fin // 2), _row_spec(fin // 2),
                  _row_spec(128), _row_spec(128),
                  _full_spec(fin, fout), _full_spec(fin, fout),
                  _full_spec(1, fout)],
        out_specs=[_row_spec(fout), _row_spec(fout // 2), _row_spec(fout // 2)],
        out_shape=[jax.ShapeDtypeStruct((N, fout), jnp.float32),
                   jax.ShapeDtypeStruct((N, fout // 2), jnp.float32),
                   jax.ShapeDtypeStruct((N, fout // 2), jnp.float32)],
    )(s_prev, agga, aggb, dega, degb, ws, wn, b.reshape(1, fout))


def _mm_last(s_prev, agga, aggb, dega, degb, ws, wn_pad, b, fin, fout):
    return pl.pallas_call(
        _mm_last_body,
        grid=(GRID,),
        in_specs=[_row_spec(fin), _row_spec(fin // 2), _row_spec(fin // 2),
                  _row_spec(128), _row_spec(128),
                  _full_spec(fin, fout), _full_spec(fin, 128),
                  _full_spec(1, fout)],
        out_specs=[_row_spec(fout), _row_spec(128)],
        out_shape=[jax.ShapeDtypeStruct((N, fout), jnp.float32),
                   jax.ShapeDtypeStruct((N, 128), jnp.float32)],
    )(s_prev, agga, aggb, dega, degb, ws, wn_pad, b.reshape(1, fout))


def _final(s, agga, aggb, dega, degb, fout):
    return pl.pallas_call(
        _final_body,
        grid=(GRID,),
        in_specs=[_row_spec(fout), _row_spec(128), _row_spec(128),
                  _row_spec(128), _row_spec(128)],
        out_specs=_row_spec(fout),
        out_shape=jax.ShapeDtypeStruct((N, fout), jnp.float32),
    )(s, agga, aggb, dega, degb)


def _fill_zero(buf, nrows, ncols):
    @pl.loop(0, nrows)
    def _(r):
        for jj in range(ncols // 16):
            buf[r, pl.ds(jj * 16, 16)] = jnp.zeros((16,), jnp.float32)


def _pieces(n, cap):
    off = 0
    out = []
    while off < n:
        m = min(cap, n - off)
        out.append((off, m))
        off += m
    return out


def _each_rows(s, fn):
    @pl.when(s < 15)
    def _():
        fn(pl.multiple_of(s * RPT, 8), RPT)

    @pl.when(s == 15)
    def _():
        fn(15 * RPT, RPT_LAST)


def _zero_spmem(s, sp, stage, cap):
    def z(base, n):
        for off, m in _pieces(n, cap):
            pltpu.sync_copy(stage.at[pl.ds(0, m)], sp.at[pl.ds(base + off, m)])
    _each_rows(s, z)


def _spmem_to_hbm(s, sp, hbm, stage, cap):
    def w(base, n):
        for off, m in _pieces(n, cap):
            pltpu.sync_copy(sp.at[pl.ds(base + off, m)], stage.at[pl.ds(0, m)])
            pltpu.sync_copy(stage.at[pl.ds(0, m)], hbm.at[pl.ds(base + off, m)])
    _each_rows(s, w)


@functools.lru_cache(maxsize=None)
def _make_segsum(fh):
    mesh = plsc.VectorSubcoreMesh(core_axis_name="c", subcore_axis_name="s")
    out_type = [jax.ShapeDtypeStruct((N, fh), jnp.float32),
                jax.ShapeDtypeStruct((N, fh), jnp.float32)]
    scratch = [
        pltpu.VMEM((GC, K), jnp.int32),
        pltpu.VMEM((GC, K), jnp.int32),
        pltpu.VMEM((K, fh), jnp.float32),
        pltpu.VMEM((K, fh), jnp.float32),
        pltpu.VMEM_SHARED((N, fh), jnp.float32),
        pltpu.SemaphoreType.DMA,
        pltpu.SemaphoreType.DMA,
    ]

    def body(pa_hbm, pb_hbm, src_hbm, dst_hbm, dummy_hbm,
             agga_hbm, aggb_hbm, src_v, dst_v, r0, r1, acc, sg0, sg1):
        c = lax.axis_index("c")
        s = lax.axis_index("s")

        _fill_zero(r0, K, fh)
        _zero_spmem(s, acc, r0, 80)
        plsc.subcore_barrier()

        @pl.loop(0, NG)
        def _(g):
            pltpu.sync_copy(src_hbm.at[s, g], src_v)
            pltpu.sync_copy(dst_hbm.at[s, g], dst_v)

            def inner(p_hbm):
                def gwait(buf, sem):
                    pltpu.make_async_copy(dummy_hbm, buf, sem).wait()

                pltpu.async_copy(p_hbm.at[src_v.at[0]], r0, sg0)

                @pl.loop(0, GC, step=2)
                def _(j):
                    gwait(r0, sg0)
                    pltpu.async_copy(p_hbm.at[src_v.at[j + 1]], r1, sg1)
                    pltpu.sync_copy(r0, acc.at[dst_v.at[j]], add=True)
                    gwait(r1, sg1)

                    @pl.when(j + 2 < GC)
                    def _():
                        pltpu.async_copy(p_hbm.at[src_v.at[j + 2]], r0, sg0)

                    pltpu.sync_copy(r1, acc.at[dst_v.at[j + 1]], add=True)

            @pl.when(c == 0)
            def _():
                inner(pa_hbm)

            @pl.when(c == 1)
            def _():
                inner(pb_hbm)

        plsc.subcore_barrier()

        @pl.when(c == 0)
        def _():
            _spmem_to_hbm(s, acc, agga_hbm, r0, 80)

        @pl.when(c == 1)
        def _():
            _spmem_to_hbm(s, acc, aggb_hbm, r0, 80)

    return pl.kernel(body, out_type=out_type, mesh=mesh, scratch_types=scratch)


@functools.lru_cache(maxsize=None)
def _make_deg():
    mesh = plsc.VectorSubcoreMesh(core_axis_name="c", subcore_axis_name="s")
    out_type = [jax.ShapeDtypeStruct((N, 128), jnp.float32),
                jax.ShapeDtypeStruct((N, 128), jnp.float32)]
    scratch = [
        pltpu.VMEM((GC2, K2), jnp.int32),
        pltpu.VMEM((K2, 128), jnp.float32),
        pltpu.VMEM_SHARED((N, 128), jnp.float32),
        pltpu.SemaphoreType.DMA,
    ]

    def body(dst_hbm, dummy_hbm, dega_hbm, degb_hbm, dst_v, ones_v, dacc, ss):
        c = lax.axis_index("c")
        s = lax.axis_index("s")

        _fill_zero(ones_v, K2, 128)
        _zero_spmem(s, dacc, ones_v, 80)

        @pl.loop(0, K2)
        def _(r):
            for jj in range(8):
                ones_v[r, pl.ds(jj * 16, 16)] = jnp.ones((16,), jnp.float32)

        plsc.subcore_barrier()

        @pl.loop(0, NG2)
        def _(g):
            pltpu.sync_copy(dst_hbm.at[c, s, g], dst_v)

            @pl.loop(0, GC2)
            def _(j):
                pltpu.async_copy(ones_v, dacc.at[dst_v.at[j]], ss, add=True)

            @pl.loop(0, GC2)
            def _(j):
                pltpu.make_async_copy(dummy_hbm, ones_v, ss).wait()

        plsc.subcore_barrier()

        @pl.when(c == 0)
        def _():
            _spmem_to_hbm(s, dacc, dega_hbm, ones_v, 80)

        @pl.when(c == 1)
        def _():
            _spmem_to_hbm(s, dacc, degb_hbm, ones_v, 80)

    return pl.kernel(body, out_type=out_type, mesh=mesh, scratch_types=scratch)


K2 = 125
GC2 = 20
NG2 = E // (2 * NSUB * K2 * GC2)


@functools.lru_cache(maxsize=None)
def _make_segsum_split():
    mesh = plsc.VectorSubcoreMesh(core_axis_name="c", subcore_axis_name="s")
    out_type = [jax.ShapeDtypeStruct((N, 128), jnp.float32),
                jax.ShapeDtypeStruct((N, 128), jnp.float32)]
    scratch = [
        pltpu.VMEM((GC2, K2), jnp.int32),
        pltpu.VMEM((GC2, K2), jnp.int32),
        pltpu.VMEM((K2, 128), jnp.float32),
        pltpu.VMEM((K2, 128), jnp.float32),
        pltpu.VMEM_SHARED((N, 128), jnp.float32),
        pltpu.SemaphoreType.DMA,
        pltpu.SemaphoreType.DMA,
    ]

    def body(p_hbm, src_hbm, dst_hbm, dummy_hbm,
             agga_hbm, aggb_hbm, src_v, dst_v, r0, r1, acc, sg0, sg1):
        c = lax.axis_index("c")
        s = lax.axis_index("s")

        _fill_zero(r0, K2, 128)
        _zero_spmem(s, acc, r0, 80)
        plsc.subcore_barrier()

        @pl.loop(0, NG2)
        def _(g):
            pltpu.sync_copy(src_hbm.at[c, s, g], src_v)
            pltpu.sync_copy(dst_hbm.at[c, s, g], dst_v)

            def gwait(buf, sem):
                pltpu.make_async_copy(dummy_hbm, buf, sem).wait()

            pltpu.async_copy(p_hbm.at[src_v.at[0]], r0, sg0)

            @pl.loop(0, GC2, step=2)
            def _(j):
                gwait(r0, sg0)
                pltpu.async_copy(p_hbm.at[src_v.at[j + 1]], r1, sg1)
                pltpu.sync_copy(r0, acc.at[dst_v.at[j]], add=True)
                gwait(r1, sg1)

                @pl.when(j + 2 < GC2)
                def _():
                    pltpu.async_copy(p_hbm.at[src_v.at[j + 2]], r0, sg0)

                pltpu.sync_copy(r1, acc.at[dst_v.at[j + 1]], add=True)

        plsc.subcore_barrier()

        @pl.when(c == 0)
        def _():
            _spmem_to_hbm(s, acc, agga_hbm, r0, 80)

        @pl.when(c == 1)
        def _():
            _spmem_to_hbm(s, acc, aggb_hbm, r0, 80)

    return pl.kernel(body, out_type=out_type, mesh=mesh, scratch_types=scratch)


def kernel(x, edge_index, W_self0, W_neigh0, b0, W_self1, W_neigh1, b1,
           W_self2, W_neigh2, b2):
    src = edge_index[0]
    dst = edge_index[1]
    src2 = src.reshape(NSUB, NG, GC, K)
    dst2 = dst.reshape(NSUB, NG, GC, K)
    src5 = src.reshape(2, NSUB, NG2, GC2, K2)
    dst5 = dst.reshape(2, NSUB, NG2, GC2, K2)
    wn2_pad = jnp.pad(W_neigh2, ((0, 0), (0, 128 - C)))

    dummy = jnp.zeros((K, 128), jnp.float32)
    dega, degb = _make_deg()(dst5, dummy)
    s0, p0a, p0b = _mm_first(x, W_self0, W_neigh0, b0, H)
    agg0a, agg0b = _make_segsum(H // 2)(p0a, p0b, src2, dst2, dummy)
    s1, p1a, p1b = _mm_mid(s0, agg0a, agg0b, dega, degb,
                           W_self1, W_neigh1, b1, H, H)
    agg1a, agg1b = _make_segsum(H // 2)(p1a, p1b, src2, dst2, dummy)
    s2, p2 = _mm_last(s1, agg1a, agg1b, dega, degb, W_self2, wn2_pad, b2, H, C)
    agg2a, agg2b = _make_segsum_split()(p2, src5, dst5, dummy)
    return _final(s2, agg2a, agg2b, dega, degb, C)


# speedup vs baseline: 7.8928x; 1.0021x over previous
import functools

import jax
import jax.numpy as jnp
from jax import lax
from jax.experimental import pallas as pl
from jax.experimental.pallas import tpu as pltpu
from jax.experimental.pallas import tpu_sc as plsc

N = 10000
E = 160000
D = 256
H = 256
C = 64

NSUB = 16
K = 125
CHUNKS = E // (K * NSUB)
GC = 20
NG = CHUNKS // GC
RPT = 624
RPT_LAST = N - 15 * RPT

BR = 2000
GRID = N // BR


def _mm_first_body(x_ref, ws_ref, wn_ref, b_ref, s_ref, pa_ref, pb_ref):
    h = x_ref[...]
    s_ref[...] = jnp.dot(h, ws_ref[...], preferred_element_type=jnp.float32) + b_ref[...]
    p = jnp.dot(h, wn_ref[...], preferred_element_type=jnp.float32)
    f2 = p.shape[1] // 2
    pa_ref[...] = p[:, :f2]
    pb_ref[...] = p[:, f2:]


def _mm_mid_body(s_prev_ref, agga_ref, aggb_ref, dega_ref, degb_ref,
                 ws_ref, wn_ref, b_ref, s_ref, pa_ref, pb_ref):
    inv = 1.0 / jnp.maximum(dega_ref[...][:, :1] + degb_ref[...][:, :1], 1.0)
    h = s_prev_ref[...] + jnp.concatenate(
        [agga_ref[...] * inv, aggb_ref[...] * inv], axis=1)
    h = jnp.maximum(h, 0.0)
    s_ref[...] = jnp.dot(h, ws_ref[...], preferred_element_type=jnp.float32) + b_ref[...]
    p = jnp.dot(h, wn_ref[...], preferred_element_type=jnp.float32)
    f2 = p.shape[1] // 2
    pa_ref[...] = p[:, :f2]
    pb_ref[...] = p[:, f2:]


def _mm_last_body(s_prev_ref, agga_ref, aggb_ref, dega_ref, degb_ref,
                  ws_ref, wn_ref, b_ref, s_ref, p_ref):
    inv = 1.0 / jnp.maximum(dega_ref[...][:, :1] + degb_ref[...][:, :1], 1.0)
    h = s_prev_ref[...] + jnp.concatenate(
        [agga_ref[...] * inv, aggb_ref[...] * inv], axis=1)
    h = jnp.maximum(h, 0.0)
    s_ref[...] = jnp.dot(h, ws_ref[...], preferred_element_type=jnp.float32) + b_ref[...]
    p_ref[...] = jnp.dot(h, wn_ref[...], preferred_element_type=jnp.float32)


def _final_body(s_ref, agga_ref, aggb_ref, dega_ref, degb_ref, o_ref):
    inv = 1.0 / jnp.maximum(dega_ref[...][:, :1] + degb_ref[...][:, :1], 1.0)
    o_ref[...] = s_ref[...] + (agga_ref[...][:, :C] + aggb_ref[...][:, :C]) * inv


def _row_spec(w):
    return pl.BlockSpec((BR, w), lambda i: (i, 0))


def _full_spec(r, c):
    return pl.BlockSpec((r, c), lambda i: (0, 0))


def _mm_first(x, ws, wn, b, fout):
    return pl.pallas_call(
        _mm_first_body,
        grid=(GRID,),
        in_specs=[_row_spec(D), _full_spec(D, fout), _full_spec(D, fout),
                  _full_spec(1, fout)],
        out_specs=[_row_spec(fout), _row_spec(fout // 2), _row_spec(fout // 2)],
        out_shape=[jax.ShapeDtypeStruct((N, fout), jnp.float32),
                   jax.ShapeDtypeStruct((N, fout // 2), jnp.float32),
                   jax.ShapeDtypeStruct((N, fout // 2), jnp.float32)],
    )(x, ws, wn, b.reshape(1, fout))


def _mm_mid(s_prev, agga, aggb, dega, degb, ws, wn, b, fin, fout):
    return pl.pallas_call(
        _mm_mid_body,
        grid=(GRID,),
        in_specs=[_row_spec(fin), _row_spec(fin // 2), _row_spec(fin // 2),
                  _row_spec(128), _row_spec(128),
                  _full_spec(fin, fout), _full_spec(fin, fout),
                  _full_spec(1, fout)],
        out_specs=[_row_spec(fout), _row_spec(fout // 2), _row_spec(fout // 2)],
        out_shape=[jax.ShapeDtypeStruct((N, fout), jnp.float32),
                   jax.ShapeDtypeStruct((N, fout // 2), jnp.float32),
                   jax.ShapeDtypeStruct((N, fout // 2), jnp.float32)],
    )(s_prev, agga, aggb, dega, degb, ws, wn, b.reshape(1, fout))


def _mm_last(s_prev, agga, aggb, dega, degb, ws, wn_pad, b, fin, fout):
    return pl.pallas_call(
        _mm_last_body,
        grid=(GRID,),
        in_specs=[_row_spec(fin), _row_spec(fin // 2), _row_spec(fin // 2),
                  _row_spec(128), _row_spec(128),
                  _full_spec(fin, fout), _full_spec(fin, 128),
                  _full_spec(1, fout)],
        out_specs=[_row_spec(fout), _row_spec(128)],
        out_shape=[jax.ShapeDtypeStruct((N, fout), jnp.float32),
                   jax.ShapeDtypeStruct((N, 128), jnp.float32)],
    )(s_prev, agga, aggb, dega, degb, ws, wn_pad, b.reshape(1, fout))


def _final(s, agga, aggb, dega, degb, fout):
    return pl.pallas_call(
        _final_body,
        grid=(GRID,),
        in_specs=[_row_spec(fout), _row_spec(128), _row_spec(128),
                  _row_spec(128), _row_spec(128)],
        out_specs=_row_spec(fout),
        out_shape=jax.ShapeDtypeStruct((N, fout), jnp.float32),
    )(s, agga, aggb, dega, degb)


def _fill_zero(buf, nrows, ncols):
    @pl.loop(0, nrows)
    def _(r):
        for jj in range(ncols // 16):
            buf[r, pl.ds(jj * 16, 16)] = jnp.zeros((16,), jnp.float32)


def _pieces(n, cap):
    off = 0
    out = []
    while off < n:
        m = min(cap, n - off)
        out.append((off, m))
        off += m
    return out


def _each_rows(s, fn):
    @pl.when(s < 15)
    def _():
        fn(pl.multiple_of(s * RPT, 8), RPT)

    @pl.when(s == 15)
    def _():
        fn(15 * RPT, RPT_LAST)


def _zero_spmem(s, sp, stage, cap):
    def z(base, n):
        for off, m in _pieces(n, cap):
            pltpu.sync_copy(stage.at[pl.ds(0, m)], sp.at[pl.ds(base + off, m)])
    _each_rows(s, z)


def _spmem_to_hbm(s, sp, hbm, stage, cap):
    def w(base, n):
        for off, m in _pieces(n, cap):
            pltpu.sync_copy(sp.at[pl.ds(base + off, m)], stage.at[pl.ds(0, m)])
            pltpu.sync_copy(stage.at[pl.ds(0, m)], hbm.at[pl.ds(base + off, m)])
    _each_rows(s, w)


@functools.lru_cache(maxsize=None)
def _make_segsum(fh):
    mesh = plsc.VectorSubcoreMesh(core_axis_name="c", subcore_axis_name="s")
    out_type = [jax.ShapeDtypeStruct((N, fh), jnp.float32),
                jax.ShapeDtypeStruct((N, fh), jnp.float32)]
    scratch = [
        pltpu.VMEM((GC, K), jnp.int32),
        pltpu.VMEM((GC, K), jnp.int32),
        pltpu.VMEM((K, fh), jnp.float32),
        pltpu.VMEM((K, fh), jnp.float32),
        pltpu.VMEM_SHARED((N, fh), jnp.float32),
        pltpu.SemaphoreType.DMA,
        pltpu.SemaphoreType.DMA,
        pltpu.SemaphoreType.DMA,
        pltpu.SemaphoreType.DMA,
    ]

    def body(pa_hbm, pb_hbm, src_hbm, dst_hbm, dummy_hbm,
             agga_hbm, aggb_hbm, src_v, dst_v, r0, r1, acc,
             sg0, sg1, ss0, ss1):
        c = lax.axis_index("c")
        s = lax.axis_index("s")

        _fill_zero(r0, K, fh)
        _zero_spmem(s, acc, r0, 80)
        plsc.subcore_barrier()

        @pl.loop(0, NG)
        def _(g):
            pltpu.sync_copy(src_hbm.at[s, g], src_v)
            pltpu.sync_copy(dst_hbm.at[s, g], dst_v)

            def inner(p_hbm):
                def dwait(buf, sem):
                    pltpu.make_async_copy(dummy_hbm, buf, sem).wait()

                pltpu.async_copy(p_hbm.at[src_v.at[0]], r0, sg0)

                @pl.loop(0, GC, step=2)
                def _(j):
                    dwait(r0, sg0)

                    @pl.when(j > 0)
                    def _():
                        dwait(r1, ss1)

                    pltpu.async_copy(p_hbm.at[src_v.at[j + 1]], r1, sg1)
                    pltpu.async_copy(r0, acc.at[dst_v.at[j]], ss0, add=True)
                    dwait(r1, sg1)

                    @pl.when(j + 2 < GC)
                    def _():
                        dwait(r0, ss0)
                        pltpu.async_copy(p_hbm.at[src_v.at[j + 2]], r0, sg0)

                    pltpu.async_copy(r1, acc.at[dst_v.at[j + 1]], ss1, add=True)

                dwait(r0, ss0)
                dwait(r1, ss1)

            @pl.when(c == 0)
            def _():
                inner(pa_hbm)

            @pl.when(c == 1)
            def _():
                inner(pb_hbm)

        plsc.subcore_barrier()

        @pl.when(c == 0)
        def _():
            _spmem_to_hbm(s, acc, agga_hbm, r0, 80)

        @pl.when(c == 1)
        def _():
            _spmem_to_hbm(s, acc, aggb_hbm, r0, 80)

    return pl.kernel(body, out_type=out_type, mesh=mesh, scratch_types=scratch)


@functools.lru_cache(maxsize=None)
def _make_deg():
    mesh = plsc.VectorSubcoreMesh(core_axis_name="c", subcore_axis_name="s")
    out_type = [jax.ShapeDtypeStruct((N, 128), jnp.float32),
                jax.ShapeDtypeStruct((N, 128), jnp.float32)]
    scratch = [
        pltpu.VMEM((GC2, K2), jnp.int32),
        pltpu.VMEM((K2, 128), jnp.float32),
        pltpu.VMEM_SHARED((N, 128), jnp.float32),
        pltpu.SemaphoreType.DMA,
    ]

    def body(dst_hbm, dummy_hbm, dega_hbm, degb_hbm, dst_v, ones_v, dacc, ss):
        c = lax.axis_index("c")
        s = lax.axis_index("s")

        _fill_zero(ones_v, K2, 128)
        _zero_spmem(s, dacc, ones_v, 80)

        @pl.loop(0, K2)
        def _(r):
            for jj in range(8):
                ones_v[r, pl.ds(jj * 16, 16)] = jnp.ones((16,), jnp.float32)

        plsc.subcore_barrier()

        @pl.loop(0, NG2)
        def _(g):
            pltpu.sync_copy(dst_hbm.at[c, s, g], dst_v)

            @pl.loop(0, GC2)
            def _(j):
                pltpu.async_copy(ones_v, dacc.at[dst_v.at[j]], ss, add=True)

            @pl.loop(0, GC2)
            def _(j):
                pltpu.make_async_copy(dummy_hbm, ones_v, ss).wait()

        plsc.subcore_barrier()

        @pl.when(c == 0)
        def _():
            _spmem_to_hbm(s, dacc, dega_hbm, ones_v, 80)

        @pl.when(c == 1)
        def _():
            _spmem_to_hbm(s, dacc, degb_hbm, ones_v, 80)

    return pl.kernel(body, out_type=out_type, mesh=mesh, scratch_types=scratch)


K2 = 125
GC2 = 20
NG2 = E // (2 * NSUB * K2 * GC2)


@functools.lru_cache(maxsize=None)
def _make_segsum_split():
    mesh = plsc.VectorSubcoreMesh(core_axis_name="c", subcore_axis_name="s")
    out_type = [jax.ShapeDtypeStruct((N, 128), jnp.float32),
                jax.ShapeDtypeStruct((N, 128), jnp.float32)]
    scratch = [
        pltpu.VMEM((GC2, K2), jnp.int32),
        pltpu.VMEM((GC2, K2), jnp.int32),
        pltpu.VMEM((K2, 128), jnp.float32),
        pltpu.VMEM((K2, 128), jnp.float32),
        pltpu.VMEM_SHARED((N, 128), jnp.float32),
        pltpu.SemaphoreType.DMA,
        pltpu.SemaphoreType.DMA,
    ]

    def body(p_hbm, src_hbm, dst_hbm, dummy_hbm,
             agga_hbm, aggb_hbm, src_v, dst_v, r0, r1, acc, sg0, sg1):
        c = lax.axis_index("c")
        s = lax.axis_index("s")

        _fill_zero(r0, K2, 128)
        _zero_spmem(s, acc, r0, 80)
        plsc.subcore_barrier()

        @pl.loop(0, NG2)
        def _(g):
            pltpu.sync_copy(src_hbm.at[c, s, g], src_v)
            pltpu.sync_copy(dst_hbm.at[c, s, g], dst_v)

            def gwait(buf, sem):
                pltpu.make_async_copy(dummy_hbm, buf, sem).wait()

            pltpu.async_copy(p_hbm.at[src_v.at[0]], r0, sg0)

            @pl.loop(0, GC2, step=2)
            def _(j):
                gwait(r0, sg0)
                pltpu.async_copy(p_hbm.at[src_v.at[j + 1]], r1, sg1)
                pltpu.sync_copy(r0, acc.at[dst_v.at[j]], add=True)
                gwait(r1, sg1)

                @pl.when(j + 2 < GC2)
                def _():
                    pltpu.async_copy(p_hbm.at[src_v.at[j + 2]], r0, sg0)

                pltpu.sync_copy(r1, acc.at[dst_v.at[j + 1]], add=True)

        plsc.subcore_barrier()

        @pl.when(c == 0)
        def _():
            _spmem_to_hbm(s, acc, agga_hbm, r0, 80)

        @pl.when(c == 1)
        def _():
            _spmem_to_hbm(s, acc, aggb_hbm, r0, 80)

    return pl.kernel(body, out_type=out_type, mesh=mesh, scratch_types=scratch)


def kernel(x, edge_index, W_self0, W_neigh0, b0, W_self1, W_neigh1, b1,
           W_self2, W_neigh2, b2):
    src = edge_index[0]
    dst = edge_index[1]
    src2 = src.reshape(NSUB, NG, GC, K)
    dst2 = dst.reshape(NSUB, NG, GC, K)
    src5 = src.reshape(2, NSUB, NG2, GC2, K2)
    dst5 = dst.reshape(2, NSUB, NG2, GC2, K2)
    wn2_pad = jnp.pad(W_neigh2, ((0, 0), (0, 128 - C)))

    dummy = jnp.zeros((K, 128), jnp.float32)
    dega, degb = _make_deg()(dst5, dummy)
    s0, p0a, p0b = _mm_first(x, W_self0, W_neigh0, b0, H)
    agg0a, agg0b = _make_segsum(H // 2)(p0a, p0b, src2, dst2, dummy)
    s1, p1a, p1b = _mm_mid(s0, agg0a, agg0b, dega, degb,
                           W_self1, W_neigh1, b1, H, H)
    agg1a, agg1b = _make_segsum(H // 2)(p1a, p1b, src2, dst2, dummy)
    s2, p2 = _mm_last(s1, agg1a, agg1b, dega, degb, W_self2, wn2_pad, b2, H, C)
    agg2a, agg2b = _make_segsum_split()(p2, src5, dst5, dummy)
    return _final(s2, agg2a, agg2b, dega, degb, C)
